# baseline (speedup 1.0000x reference)
import functools

import jax
import jax.numpy as jnp
from jax import lax
from jax.experimental import pallas as pl
from jax.experimental.pallas import tpu as pltpu
from jax.experimental.pallas import tpu_sc as plsc

EMBED_DIM = 32
BATCH = 16384
FIELDS = 26
NUM_CORES = 2
NUM_SUBCORES = 16
NW = NUM_CORES * NUM_SUBCORES
TASK_B = 1024
CT_PER_TASK = TASK_B // 128
NTASK = (BATCH // TASK_B) * FIELDS
TASKS_PER_W = NTASK // NW
CTC_PER_F = BATCH // TASK_B
SPAD = 131

_mesh = plsc.VectorSubcoreMesh(core_axis_name="c", subcore_axis_name="s")

VOCAB = 1000000
NTC = VOCAB // 128
TAIL0 = NTC * 128


KCOL = 8
NCHK = 976
CPW = 31
IPAD = KCOL * 128 + 1


@functools.partial(
    pl.kernel,
    mesh=_mesh,
    out_type=jax.ShapeDtypeStruct((VOCAB * EMBED_DIM // 128, 128), jnp.float32),
    scratch_types=[
        pltpu.VMEM((2, 32, IPAD), jnp.float32),
        pltpu.VMEM((2, 128, 128), jnp.float32),
        pltpu.VMEM((16, 128), jnp.float32),
        pltpu.SemaphoreType.DMA,
        pltpu.SemaphoreType.DMA,
        pltpu.SemaphoreType.DMA,
        pltpu.SemaphoreType.DMA,
    ],
    compiler_params=pltpu.CompilerParams(use_tc_tiling_on_sc=True,
                                         needs_layout_passes=False),
)
def _format_kernel(tabt_hbm, tail_hbm, out_hbm, inb, stg, tailb,
                   isem0, isem1, osem0, osem1):
    wid = lax.axis_index("s") * NUM_CORES + lax.axis_index("c")
    iota = lax.iota(jnp.int32, 16)
    iotas = [iota, iota + 16]
    zero16 = jnp.zeros((16,), jnp.int32)
    isems = [isem0, isem1]
    osems = [osem0, osem1]
    s_w = (wid * (NCHK - CPW)) // (NW - 1)

    def in_copy(c, b):
        return pltpu.make_async_copy(
            tabt_hbm.at[:, pl.ds(c * (KCOL * 128), KCOL * 128)],
            inb.at[b, :, pl.ds(0, KCOL * 128)], isems[b])

    def out_copy(c, h):
        return pltpu.make_async_copy(
            stg.at[h], out_hbm.at[pl.ds(c * (KCOL * 32) + h * 128, 128)],
            osems[h])

    def transpose_half(b, h):
        for jj in range(KCOL // 2):
            j = h * (KCOL // 2) + jj

            @plsc.parallel_loop(0, 32, unroll=16)
            def row_body(r_o):
                cbase = j * 128 + 4 * r_o
                cvecs = [zero16 + (cbase + q) for q in range(4)]
                for c00 in range(0, 128, 16):
                    vals = plsc.load_gather(
                        inb.at[b],
                        [iotas[(c00 // 16) & 1], cvecs[c00 // 32]])
                    stg[h, jj * 32 + r_o, pl.ds(c00, 16)] = vals

    def do_chunk(c, b):
        in_copy(c, b).wait()
        for h in range(2):
            @pl.when(c > s_w)
            def _drain():
                out_copy(c - 1, h).wait()
            transpose_half(b, h)
            out_copy(c, h).start()

    in_copy(s_w, 0).start()

    def pair_body(p, _):
        c = s_w + 2 * p
        in_copy(c + 1, 1).start()
        do_chunk(c, 0)
        in_copy(c + 2, 0).start()
        do_chunk(c + 1, 1)
        return 0
    lax.fori_loop(0, (CPW - 1) // 2, pair_body, 0)

    last = s_w + CPW - 1
    do_chunk(last, 0)
    out_copy(last, 0).wait()
    out_copy(last, 1).wait()

    @pl.when(wid == NW - 1)
    def _leftover():
        pltpu.sync_copy(tabt_hbm.at[:, pl.ds(NCHK * KCOL * 128, 512)],
                        inb.at[1, :, pl.ds(0, 512)])
        for j in range(4):
            @plsc.parallel_loop(0, 32, unroll=16)
            def lo_body(r_o):
                cbase = j * 128 + 4 * r_o
                cvecs = [zero16 + (cbase + q) for q in range(4)]
                for c00 in range(0, 128, 16):
                    vals = plsc.load_gather(
                        inb.at[1],
                        [iotas[(c00 // 16) & 1], cvecs[c00 // 32]])
                    stg[1, j * 32 + r_o, pl.ds(c00, 16)] = vals
        pltpu.sync_copy(stg.at[1],
                        out_hbm.at[pl.ds(NCHK * KCOL * 32, 128)])
        pltpu.sync_copy(tail_hbm, tailb)
        pltpu.sync_copy(tailb, out_hbm.at[pl.ds(TAIL0 * EMBED_DIM // 128, 16)])


@functools.partial(
    pl.kernel,
    mesh=_mesh,
    out_type=jax.ShapeDtypeStruct((FIELDS, EMBED_DIM // 8, BATCH // 128, 8, 128),
                                  jnp.float32),
    scratch_types=[
        pltpu.VMEM((2, TASK_B), jnp.int32),
        pltpu.VMEM((2, TASK_B, EMBED_DIM), jnp.float32),
        pltpu.VMEM((CT_PER_TASK, 8, SPAD), jnp.float32),
        pltpu.VMEM((CT_PER_TASK, 8, SPAD), jnp.float32),
        pltpu.VMEM((CT_PER_TASK, 8, SPAD), jnp.float32),
        pltpu.VMEM((CT_PER_TASK, 8, SPAD), jnp.float32),
        pltpu.SemaphoreType.DMA,
        pltpu.SemaphoreType.DMA,
        pltpu.SemaphoreType.DMA,
    ],
    compiler_params=pltpu.CompilerParams(use_tc_tiling_on_sc=False,
                                         needs_layout_passes=False),
)
def _gather_kernel(idx_hbm, table_hbm, out_hbm, idx_v, gbuf,
                   st0, st1, st2, st3, gsem0, gsem1, wsem):
    wid = lax.axis_index("s") * NUM_CORES + lax.axis_index("c")
    iota = lax.iota(jnp.int32, 16)
    rvec = iota & 7
    m_lo = iota < 8
    m_hi = iota >= 8
    gsems = [gsem0, gsem1]
    stages = [st0, st1, st2, st3]

    def load_idx_and_gather(tl, buf):
        t = wid * TASKS_PER_W + tl
        f = t // CTC_PER_F
        b0 = (t % CTC_PER_F) * TASK_B
        pltpu.sync_copy(idx_hbm.at[f, pl.ds(b0, TASK_B)], idx_v.at[buf])
        return pltpu.async_copy(table_hbm.at[idx_v.at[buf]],
                                gbuf.at[buf], gsems[buf])

    gathers = [None, None]
    gathers[0] = load_idx_and_gather(0, 0)
    for tl in range(TASKS_PER_W):
        buf = tl % 2
        nbuf = (tl + 1) % 2
        gathers[buf].wait()
        if tl + 1 < TASKS_PER_W:
            gathers[nbuf] = load_idx_and_gather(tl + 1, nbuf)

        t = wid * TASKS_PER_W + tl
        f = t // CTC_PER_F
        ct0 = (t % CTC_PER_F) * CT_PER_TASK

        @plsc.parallel_loop(0, CT_PER_TASK * 8, unroll=2)
        def c16_body(i):
            ctl = i // 8
            cbase = (i % 8) * 16
            d0 = jnp.zeros((16,), jnp.int32) + ctl
            b_base = ctl * 128 + cbase
            for bi in range(16):
                d2 = jnp.zeros((16,), jnp.int32) + (cbase + bi)
                v0 = gbuf[buf, b_base + bi, pl.ds(0, 16)]
                v1 = gbuf[buf, b_base + bi, pl.ds(16, 16)]
                plsc.store_scatter(st0, [d0, rvec, d2], v0, mask=m_lo)
                plsc.store_scatter(st1, [d0, rvec, d2], v0, mask=m_hi)
                plsc.store_scatter(st2, [d0, rvec, d2], v1, mask=m_lo)
                plsc.store_scatter(st3, [d0, rvec, d2], v1, mask=m_hi)

        for blk in range(4):
            pltpu.async_copy(stages[blk].at[:, :, pl.ds(0, 128)],
                             out_hbm.at[f, blk, pl.ds(ct0, CT_PER_TASK)],
                             wsem).wait()


def kernel(pokemon_state, table):
    idx_t = pokemon_state.T.astype(jnp.int32)
    tail = table[TAIL0:].reshape(16, 128)
    table_lin = _format_kernel(table.T, tail).reshape(VOCAB, EMBED_DIM)
    out5 = _gather_kernel(idx_t, table_lin)
    return out5.transpose(2, 4, 0, 1, 3).reshape(BATCH, FIELDS, EMBED_DIM)


# speedup vs baseline: 1.0113x; 1.0113x over previous
import functools

import jax
import jax.numpy as jnp
from jax import lax
from jax.experimental import pallas as pl
from jax.experimental.pallas import tpu as pltpu
from jax.experimental.pallas import tpu_sc as plsc

EMBED_DIM = 32
BATCH = 16384
FIELDS = 26
NUM_CORES = 2
NUM_SUBCORES = 16
NW = NUM_CORES * NUM_SUBCORES
TASK_B = 1024
CT_PER_TASK = TASK_B // 128
NTASK = (BATCH // TASK_B) * FIELDS
TASKS_PER_W = NTASK // NW
CTC_PER_F = BATCH // TASK_B
SPAD = 131

_mesh = plsc.VectorSubcoreMesh(core_axis_name="c", subcore_axis_name="s")

VOCAB = 1000000
NTC = VOCAB // 128
TAIL0 = NTC * 128


KCOL = 8
NCHK = 976
CPW = 31
IPAD = KCOL * 128 + 1


@functools.partial(
    pl.kernel,
    mesh=_mesh,
    out_type=jax.ShapeDtypeStruct((VOCAB * EMBED_DIM // 128, 128), jnp.float32),
    scratch_types=[
        pltpu.VMEM((2, 32, IPAD), jnp.float32),
        pltpu.VMEM((2, 128, 128), jnp.float32),
        pltpu.VMEM((16, 128), jnp.float32),
        pltpu.SemaphoreType.DMA,
        pltpu.SemaphoreType.DMA,
        pltpu.SemaphoreType.DMA,
        pltpu.SemaphoreType.DMA,
    ],
    compiler_params=pltpu.CompilerParams(use_tc_tiling_on_sc=True,
                                         needs_layout_passes=False),
)
def _format_kernel(tabt_hbm, tail_hbm, out_hbm, inb, stg, tailb,
                   isem0, isem1, osem0, osem1):
    wid = lax.axis_index("s") * NUM_CORES + lax.axis_index("c")
    iota = lax.iota(jnp.int32, 16)
    iotas = [iota, iota + 16]
    zero16 = jnp.zeros((16,), jnp.int32)
    isems = [isem0, isem1]
    osems = [osem0, osem1]
    s_w = (wid * (NCHK - CPW)) // (NW - 1)

    def in_copy(c, b):
        return pltpu.make_async_copy(
            tabt_hbm.at[:, pl.ds(c * (KCOL * 128), KCOL * 128)],
            inb.at[b, :, pl.ds(0, KCOL * 128)], isems[b])

    def out_copy(c, h):
        return pltpu.make_async_copy(
            stg.at[h], out_hbm.at[pl.ds(c * (KCOL * 32) + h * 128, 128)],
            osems[h])

    def transpose_half(b, h):
        for jj in range(KCOL // 2):
            j = h * (KCOL // 2) + jj

            @plsc.parallel_loop(0, 32, unroll=8)
            def row_body(r_o):
                cbase = j * 128 + 4 * r_o
                cvecs = [zero16 + (cbase + q) for q in range(4)]
                for c00 in range(0, 128, 16):
                    vals = plsc.load_gather(
                        inb.at[b],
                        [iotas[(c00 // 16) & 1], cvecs[c00 // 32]])
                    stg[h, jj * 32 + r_o, pl.ds(c00, 16)] = vals

    def do_chunk(c, b):
        in_copy(c, b).wait()
        for h in range(2):
            @pl.when(c > s_w)
            def _drain():
                out_copy(c - 1, h).wait()
            transpose_half(b, h)
            out_copy(c, h).start()

    in_copy(s_w, 0).start()

    def pair_body(p, _):
        c = s_w + 2 * p
        in_copy(c + 1, 1).start()
        do_chunk(c, 0)
        in_copy(c + 2, 0).start()
        do_chunk(c + 1, 1)
        return 0
    lax.fori_loop(0, (CPW - 1) // 2, pair_body, 0)

    last = s_w + CPW - 1
    do_chunk(last, 0)
    out_copy(last, 0).wait()
    out_copy(last, 1).wait()

    @pl.when(wid == NW - 1)
    def _leftover():
        pltpu.sync_copy(tabt_hbm.at[:, pl.ds(NCHK * KCOL * 128, 512)],
                        inb.at[1, :, pl.ds(0, 512)])
        for j in range(4):
            @plsc.parallel_loop(0, 32, unroll=8)
            def lo_body(r_o):
                cbase = j * 128 + 4 * r_o
                cvecs = [zero16 + (cbase + q) for q in range(4)]
                for c00 in range(0, 128, 16):
                    vals = plsc.load_gather(
                        inb.at[1],
                        [iotas[(c00 // 16) & 1], cvecs[c00 // 32]])
                    stg[1, j * 32 + r_o, pl.ds(c00, 16)] = vals
        pltpu.sync_copy(stg.at[1],
                        out_hbm.at[pl.ds(NCHK * KCOL * 32, 128)])
        pltpu.sync_copy(tail_hbm, tailb)
        pltpu.sync_copy(tailb, out_hbm.at[pl.ds(TAIL0 * EMBED_DIM // 128, 16)])


@functools.partial(
    pl.kernel,
    mesh=_mesh,
    out_type=jax.ShapeDtypeStruct((FIELDS, EMBED_DIM // 8, BATCH // 128, 8, 128),
                                  jnp.float32),
    scratch_types=[
        pltpu.VMEM((2, TASK_B), jnp.int32),
        pltpu.VMEM((2, TASK_B, EMBED_DIM), jnp.float32),
        pltpu.VMEM((CT_PER_TASK, 8, SPAD), jnp.float32),
        pltpu.VMEM((CT_PER_TASK, 8, SPAD), jnp.float32),
        pltpu.VMEM((CT_PER_TASK, 8, SPAD), jnp.float32),
        pltpu.VMEM((CT_PER_TASK, 8, SPAD), jnp.float32),
        pltpu.SemaphoreType.DMA,
        pltpu.SemaphoreType.DMA,
        pltpu.SemaphoreType.DMA,
    ],
    compiler_params=pltpu.CompilerParams(use_tc_tiling_on_sc=False,
                                         needs_layout_passes=False),
)
def _gather_kernel(idx_hbm, table_hbm, out_hbm, idx_v, gbuf,
                   st0, st1, st2, st3, gsem0, gsem1, wsem):
    wid = lax.axis_index("s") * NUM_CORES + lax.axis_index("c")
    iota = lax.iota(jnp.int32, 16)
    rvec = iota & 7
    m_lo = iota < 8
    m_hi = iota >= 8
    gsems = [gsem0, gsem1]
    stages = [st0, st1, st2, st3]

    def load_idx_and_gather(tl, buf):
        t = wid * TASKS_PER_W + tl
        f = t // CTC_PER_F
        b0 = (t % CTC_PER_F) * TASK_B
        pltpu.sync_copy(idx_hbm.at[f, pl.ds(b0, TASK_B)], idx_v.at[buf])
        return pltpu.async_copy(table_hbm.at[idx_v.at[buf]],
                                gbuf.at[buf], gsems[buf])

    gathers = [None, None]
    gathers[0] = load_idx_and_gather(0, 0)
    for tl in range(TASKS_PER_W):
        buf = tl % 2
        nbuf = (tl + 1) % 2
        gathers[buf].wait()
        if tl + 1 < TASKS_PER_W:
            gathers[nbuf] = load_idx_and_gather(tl + 1, nbuf)

        t = wid * TASKS_PER_W + tl
        f = t // CTC_PER_F
        ct0 = (t % CTC_PER_F) * CT_PER_TASK

        @plsc.parallel_loop(0, CT_PER_TASK * 8, unroll=2)
        def c16_body(i):
            ctl = i // 8
            cbase = (i % 8) * 16
            d0 = jnp.zeros((16,), jnp.int32) + ctl
            b_base = ctl * 128 + cbase
            for bi in range(16):
                d2 = jnp.zeros((16,), jnp.int32) + (cbase + bi)
                v0 = gbuf[buf, b_base + bi, pl.ds(0, 16)]
                v1 = gbuf[buf, b_base + bi, pl.ds(16, 16)]
                plsc.store_scatter(st0, [d0, rvec, d2], v0, mask=m_lo)
                plsc.store_scatter(st1, [d0, rvec, d2], v0, mask=m_hi)
                plsc.store_scatter(st2, [d0, rvec, d2], v1, mask=m_lo)
                plsc.store_scatter(st3, [d0, rvec, d2], v1, mask=m_hi)

        for blk in range(4):
            pltpu.async_copy(stages[blk].at[:, :, pl.ds(0, 128)],
                             out_hbm.at[f, blk, pl.ds(ct0, CT_PER_TASK)],
                             wsem).wait()


def kernel(pokemon_state, table):
    idx_t = pokemon_state.T.astype(jnp.int32)
    tail = table[TAIL0:].reshape(16, 128)
    table_lin = _format_kernel(table.T, tail).reshape(VOCAB, EMBED_DIM)
    out5 = _gather_kernel(idx_t, table_lin)
    return out5.transpose(2, 4, 0, 1, 3).reshape(BATCH, FIELDS, EMBED_DIM)
